# Initial kernel scaffold; baseline (speedup 1.0000x reference)
import jax
import jax.numpy as jnp
from jax.experimental import pallas as pl


def kernel(x, pos, batch, W1, b1, g1, be1, W2, b2, g2, be2):
    raise NotImplementedError("write your pallas kernel here")


# speedup vs baseline: 8.1593x; 8.1593x over previous
import functools

import jax
import jax.numpy as jnp
from jax import lax
from jax.experimental import pallas as pl
from jax.experimental.pallas import tpu as pltpu
from jax.experimental.pallas import tpu_sc as plsc

_RATIO = 0.5
_RADIUS = 0.2
_K = 64
_BIG = 2**30
_IP = False


def _fps_body(px_ref, py_ref, pz_ref, bf_ref, sel_ref, cx_ref, cy_ref,
              cz_ref, cb_ref, *, n, m):
    rows = px_ref.shape[0]
    sb, sl = sel_ref.shape
    fio = (lax.broadcasted_iota(jnp.int32, (rows, 128), 0) * 128
           + lax.broadcasted_iota(jnp.int32, (rows, 128), 1))
    sio = (lax.broadcasted_iota(jnp.int32, (sb, sl), 0) * sl
           + lax.broadcasted_iota(jnp.int32, (sb, sl), 1))
    valid = fio < n
    px = px_ref[...]
    py = py_ref[...]
    pz = pz_ref[...]
    bf = bf_ref[...]
    eq0 = fio == 0
    p0x = jnp.sum(jnp.where(eq0, px, 0.0))
    p0y = jnp.sum(jnp.where(eq0, py, 0.0))
    p0z = jnp.sum(jnp.where(eq0, pz, 0.0))
    b0 = jnp.sum(jnp.where(eq0, bf, 0.0))
    dists0 = jnp.where(valid, jnp.inf, -jnp.inf)
    u0 = sio == 0
    carry0 = (dists0,
              jnp.zeros((sb, sl), jnp.int32),
              jnp.where(u0, p0x, 0.0), jnp.where(u0, p0y, 0.0),
              jnp.where(u0, p0z, 0.0), jnp.where(u0, b0, 0.0),
              p0x, p0y, p0z)

    def body(i, c):
        dists, selb, cxb, cyb, czb, cbb, lx, ly, lz = c
        dx = px - lx
        dy = py - ly
        dz = pz - lz
        d2 = dx * dx + dy * dy + dz * dz
        dists = jnp.minimum(dists, d2)
        mval = jnp.max(dists)
        eqm = dists == mval
        sidx = jnp.min(jnp.where(eqm, fio, _BIG))
        eqs = fio == sidx
        nlx = jnp.sum(jnp.where(eqs, px, 0.0))
        nly = jnp.sum(jnp.where(eqs, py, 0.0))
        nlz = jnp.sum(jnp.where(eqs, pz, 0.0))
        nbv = jnp.sum(jnp.where(eqs, bf, 0.0))
        upd = sio == i
        return (dists,
                jnp.where(upd, sidx, selb),
                jnp.where(upd, nlx, cxb), jnp.where(upd, nly, cyb),
                jnp.where(upd, nlz, czb), jnp.where(upd, nbv, cbb),
                nlx, nly, nlz)

    c = lax.fori_loop(1, m, body, carry0)
    sel_ref[...] = c[1]
    cx_ref[...] = c[2]
    cy_ref[...] = c[3]
    cz_ref[...] = c[4]
    cb_ref[...] = c[5]


def _run_fps(pos, batch, n, m):
    rows = -(-n // 128)
    npad = rows * 128
    sl = 512
    sb = -(-m // sl)
    posp = jnp.pad(pos, ((0, npad - n), (0, 0)))
    bfp = jnp.pad(batch.astype(jnp.float32), (0, npad - n))
    px = posp[:, 0].reshape(rows, 128)
    py = posp[:, 1].reshape(rows, 128)
    pz = posp[:, 2].reshape(rows, 128)
    bf = bfp.reshape(rows, 128)
    outs = pl.pallas_call(
        functools.partial(_fps_body, n=n, m=m),
        out_shape=[jax.ShapeDtypeStruct((sb, sl), jnp.int32)]
        + [jax.ShapeDtypeStruct((sb, sl), jnp.float32)] * 4,
        interpret=_IP,
    )(px, py, pz, bf)
    selb, cxb, cyb, czb, cbb = outs
    sel = selb.reshape(-1)[:m]
    cx = cxb.reshape(-1)[:m]
    cy = cyb.reshape(-1)[:m]
    cz = czb.reshape(-1)[:m]
    cb = cbb.reshape(-1)[:m]
    centers = jnp.stack([cx, cy, cz], axis=1)
    return sel, centers, cb.astype(jnp.int32)


def _topk_body(cx_ref, cy_ref, cz_ref, bc_ref, px_ref, py_ref, pz_ref,
               bt_ref, nbr_ref, msk_ref, d2_ref, *, n, k, r2):
    npad = px_ref.shape[1]
    bm = cx_ref.shape[0]
    dx = cx_ref[...] - px_ref[...]
    dy = cy_ref[...] - py_ref[...]
    dz = cz_ref[...] - pz_ref[...]
    d2 = dx * dx + dy * dy + dz * dz
    fio = lax.broadcasted_iota(jnp.int32, (bm, npad), 1)
    ok = (d2 <= r2) & (fio < n) & (bc_ref[...] == bt_ref[...])
    d2_ref[...] = jnp.where(ok, d2, jnp.inf)
    kio = lax.broadcasted_iota(jnp.int32, (bm, k), 1)

    def body(s, c):
        nbrb, mskb = c
        dv = d2_ref[...]
        mrow = jnp.min(dv, axis=1, keepdims=True)
        eq = dv == mrow
        idx = jnp.min(jnp.where(eq, fio, _BIG), axis=1, keepdims=True)
        d2_ref[...] = jnp.where(eq, jnp.inf, dv)
        vld = mrow < jnp.inf
        upd = kio == s
        nbrb = jnp.where(upd, jnp.where(vld, idx, 0), nbrb)
        mskb = jnp.where(upd, vld.astype(jnp.float32), mskb)
        return nbrb, mskb

    nbrb, mskb = lax.fori_loop(
        0, k, body,
        (jnp.zeros((bm, k), jnp.int32), jnp.zeros((bm, k), jnp.float32)))
    nbr_ref[...] = nbrb
    msk_ref[...] = mskb


def _run_topk(centers, bc, pos, batch, n, m, k):
    bm = 128
    mp = -(-m // bm) * bm
    rows = -(-n // 128)
    npad = rows * 128
    posp = jnp.pad(pos, ((0, npad - n), (0, 0)))
    cxp = jnp.pad(centers[:, 0], (0, mp - m), constant_values=1e9)
    cyp = jnp.pad(centers[:, 1], (0, mp - m), constant_values=1e9)
    czp = jnp.pad(centers[:, 2], (0, mp - m), constant_values=1e9)
    bcp = jnp.pad(bc, (0, mp - m)).reshape(mp, 1)
    btp = jnp.pad(batch, (0, npad - n)).reshape(1, npad)
    grid = mp // bm
    cspec = pl.BlockSpec((bm, 1), lambda i: (i, 0))
    pspec = pl.BlockSpec((1, npad), lambda i: (0, 0))
    nbr, msk = pl.pallas_call(
        functools.partial(_topk_body, n=n, k=k, r2=_RADIUS * _RADIUS),
        grid=(grid,),
        in_specs=[cspec, cspec, cspec, cspec, pspec, pspec, pspec, pspec],
        out_specs=[pl.BlockSpec((bm, k), lambda i: (i, 0))] * 2,
        out_shape=[jax.ShapeDtypeStruct((mp, k), jnp.int32),
                   jax.ShapeDtypeStruct((mp, k), jnp.float32)],
        scratch_shapes=[pltpu.VMEM((bm, npad), jnp.float32)],
        interpret=_IP,
    )(cxp.reshape(mp, 1), cyp.reshape(mp, 1), czp.reshape(mp, 1), bcp,
      posp[:, 0].reshape(1, npad), posp[:, 1].reshape(1, npad),
      posp[:, 2].reshape(1, npad), btp)
    return nbr[:m], msk[:m]


def _sc_gather(table, idxs):
    e, d = idxs.shape[0], table.shape[1]
    ch = 128
    nchunks = e // ch
    info = plsc.get_sparse_core_info()
    nc, ns = info.num_cores, info.num_subcores
    nw = nc * ns
    mesh = plsc.VectorSubcoreMesh(core_axis_name="c", subcore_axis_name="s")

    @functools.partial(
        pl.kernel, mesh=mesh,
        out_type=jax.ShapeDtypeStruct((e, d), jnp.float32),
        scratch_types=[pltpu.VMEM((ch,), jnp.int32),
                       pltpu.VMEM((ch, d), jnp.float32),
                       pltpu.SemaphoreType.DMA])
    def gk(table_hbm, idx_hbm, out_hbm, idx_v, rows_v, sem):
        wid = lax.axis_index("s") * nc + lax.axis_index("c")
        nmine = (nchunks - wid + nw - 1) // nw

        def it(t, _):
            base = (wid + t * nw) * ch
            pltpu.sync_copy(idx_hbm.at[pl.ds(base, ch)], idx_v)
            pltpu.async_copy(table_hbm.at[idx_v], rows_v, sem).wait()
            pltpu.sync_copy(rows_v, out_hbm.at[pl.ds(base, ch)])
            return 0

        lax.fori_loop(0, nmine, it, 0, unroll=False)

    return gk(table, idxs)


def _mlp1_body(e_ref, c_ref, m_ref, w_ref, b_ref, h_ref, s1_ref, s2_ref,
               cn_ref, *, kk):
    be, dp = e_ref.shape
    g = be // kk

    @pl.when(pl.program_id(0) == 0)
    def _():
        s1_ref[...] = jnp.zeros_like(s1_ref)
        s2_ref[...] = jnp.zeros_like(s2_ref)
        cn_ref[...] = jnp.zeros_like(cn_ref)

    e3 = e_ref[...].reshape(g, kk, dp) - c_ref[0][:, None, :]
    h = jnp.dot(e3.reshape(be, dp), w_ref[...],
                preferred_element_type=jnp.float32) + b_ref[...]
    h = jnp.maximum(h, 0.0)
    h_ref[...] = h
    m3 = m_ref[...]
    hm = h * m3
    s1_ref[...] += jnp.sum(hm, axis=0, keepdims=True)
    s2_ref[...] += jnp.sum(hm * h, axis=0, keepdims=True)
    cn_ref[...] += jnp.sum(m3)


def _mlp2_body(h_ref, m_ref, sc_ref, sh_ref, w_ref, b_ref, o_ref, s1_ref,
               s2_ref, *, kk):
    be = h_ref.shape[0]

    @pl.when(pl.program_id(0) == 0)
    def _():
        s1_ref[...] = jnp.zeros_like(s1_ref)
        s2_ref[...] = jnp.zeros_like(s2_ref)

    h1n = h_ref[...] * sc_ref[...] + sh_ref[...]
    h2 = jnp.dot(h1n, w_ref[...], preferred_element_type=jnp.float32)
    h2 = jnp.maximum(h2 + b_ref[...], 0.0)
    o_ref[...] = h2
    m3 = m_ref[...]
    hm = h2 * m3
    s1_ref[...] += jnp.sum(hm, axis=0, keepdims=True)
    s2_ref[...] += jnp.sum(hm * h2, axis=0, keepdims=True)


def _final_body(h_ref, m_ref, sc_ref, sh_ref, o_ref, *, kk):
    be, do = h_ref.shape
    g = be // kk
    h2n = h_ref[...] * sc_ref[...] + sh_ref[...]
    h2n = jnp.where(m_ref[...] > 0, h2n, -jnp.inf)
    o_ref[...] = jnp.max(h2n.reshape(g, kk, do), axis=1)


def _bn_affine(s1, s2, cn, g, be, eps=1e-5):
    cnt = jnp.maximum(cn, 1.0)
    mean = s1 / cnt
    var = jnp.maximum(s2 / cnt - mean * mean, 0.0)
    scale = g / jnp.sqrt(var + eps)
    return scale, be - mean * scale


def kernel(x, pos, batch, W1, b1, g1, be1, W2, b2, g2, be2):
    n, din = x.shape
    m = int(n * _RATIO)
    k = _K
    dim = din + 3
    dp = -(-dim // 128) * 128
    do = W2.shape[1]
    e = m * k

    sel, centers, bc = _run_fps(pos, batch, n, m)
    nbr, msk = _run_topk(centers, bc, pos, batch, n, m, k)

    table = jnp.concatenate(
        [x, pos, jnp.zeros((n, dp - dim), jnp.float32)], axis=1)
    edges = _sc_gather(table, nbr.reshape(-1))

    be_blk = 512
    grid = e // be_blk
    g_blk = be_blk // k
    cpad = jnp.concatenate(
        [jnp.zeros((m, din), jnp.float32), centers,
         jnp.zeros((m, dp - dim), jnp.float32)], axis=1)
    cpad3 = cpad.reshape(grid, g_blk, dp)
    mcol = msk.reshape(e, 1)
    w1p = jnp.pad(W1, ((0, dp - dim), (0, dp - dim)))
    b1p = jnp.pad(b1, (0, dp - dim)).reshape(1, dp)
    g1p = jnp.pad(g1, (0, dp - dim))
    be1p = jnp.pad(be1, (0, dp - dim))
    w2p = jnp.pad(W2, ((0, dp - dim), (0, 0)))
    b2p = b2.reshape(1, do)

    espec = pl.BlockSpec((be_blk, dp), lambda i: (i, 0))
    cspec = pl.BlockSpec((1, g_blk, dp), lambda i: (i, 0, 0))
    mspec = pl.BlockSpec((be_blk, 1), lambda i: (i, 0))
    wspec = pl.BlockSpec((dp, dp), lambda i: (0, 0))
    rspec = pl.BlockSpec((1, dp), lambda i: (0, 0))
    w2spec = pl.BlockSpec((dp, do), lambda i: (0, 0))
    r2spec = pl.BlockSpec((1, do), lambda i: (0, 0))

    h1, s1, s2, cn = pl.pallas_call(
        functools.partial(_mlp1_body, kk=k),
        grid=(grid,),
        in_specs=[espec, cspec, mspec, wspec, rspec],
        out_specs=[espec, rspec, rspec,
                   pl.BlockSpec((1, 1), lambda i: (0, 0))],
        out_shape=[jax.ShapeDtypeStruct((e, dp), jnp.float32),
                   jax.ShapeDtypeStruct((1, dp), jnp.float32),
                   jax.ShapeDtypeStruct((1, dp), jnp.float32),
                   jax.ShapeDtypeStruct((1, 1), jnp.float32)],
        interpret=_IP,
    )(edges, cpad3, mcol, w1p, b1p)

    sc1, sh1 = _bn_affine(s1, s2, cn[0, 0], g1p.reshape(1, dp),
                          be1p.reshape(1, dp))

    h2, t1, t2 = pl.pallas_call(
        functools.partial(_mlp2_body, kk=k),
        grid=(grid,),
        in_specs=[espec, mspec, rspec, rspec, w2spec, r2spec],
        out_specs=[pl.BlockSpec((be_blk, do), lambda i: (i, 0)),
                   r2spec, r2spec],
        out_shape=[jax.ShapeDtypeStruct((e, do), jnp.float32),
                   jax.ShapeDtypeStruct((1, do), jnp.float32),
                   jax.ShapeDtypeStruct((1, do), jnp.float32)],
        interpret=_IP,
    )(h1, mcol, sc1, sh1, w2p, b2p)

    sc2, sh2 = _bn_affine(t1, t2, cn[0, 0], g2.reshape(1, do),
                          be2.reshape(1, do))

    out = pl.pallas_call(
        functools.partial(_final_body, kk=k),
        grid=(grid,),
        in_specs=[pl.BlockSpec((be_blk, do), lambda i: (i, 0)), mspec,
                  r2spec, r2spec],
        out_specs=pl.BlockSpec((g_blk, do), lambda i: (i, 0)),
        out_shape=jax.ShapeDtypeStruct((m, do), jnp.float32),
        interpret=_IP,
    )(h2, mcol, sc2, sh2)

    return (out, centers, bc, sel)
